# baseline (speedup 1.0000x reference)
import functools

import jax
import jax.numpy as jnp
from jax import lax
from jax.experimental import pallas as pl
from jax.experimental.pallas import tpu as pltpu
from jax.experimental.pallas import tpu_sc as plsc

D = 128
LANES = 16


def _transform_table(mz_table, w1t, b_row):
    v = mz_table.shape[0]
    blk = 16384

    def body(tbl_ref, w_ref, b_ref, out_ref):
        out_ref[...] = (
            jnp.dot(tbl_ref[...], w_ref[...], preferred_element_type=jnp.float32)
            + b_ref[...]
        )

    return pl.pallas_call(
        body,
        grid=(pl.cdiv(v, blk),),
        in_specs=[
            pl.BlockSpec((blk, D), lambda i: (i, 0)),
            pl.BlockSpec((D, D), lambda i: (0, 0)),
            pl.BlockSpec((1, D), lambda i: (0, 0)),
        ],
        out_specs=pl.BlockSpec((blk, D), lambda i: (i, 0)),
        out_shape=jax.ShapeDtypeStruct((v, D), jnp.float32),
    )(mz_table, w1t, b_row)


def _gather_combine(table_t, idx_flat, int_flat, w_last):
    n = idx_flat.shape[0]
    info = plsc.get_sparse_core_info()
    nc, ns = info.num_cores, info.num_subcores
    nw = nc * ns
    per_w = n // nw
    ch = 128
    nbuf = 5
    nchunk = per_w // ch
    nouter = nchunk // nbuf
    ngrp = ch // LANES

    mesh = plsc.VectorSubcoreMesh(core_axis_name="c", subcore_axis_name="s")

    @functools.partial(
        pl.kernel,
        mesh=mesh,
        out_type=jax.ShapeDtypeStruct((n, D), jnp.float32),
        scratch_types=[
            pltpu.VMEM((per_w,), jnp.int32),
            pltpu.VMEM((per_w,), jnp.float32),
            pltpu.VMEM((nbuf, ch, D), jnp.float32),
            pltpu.VMEM((D,), jnp.float32),
            pltpu.SemaphoreType.DMA((nbuf,)),
            pltpu.SemaphoreType.DMA((nbuf,)),
            pltpu.SemaphoreType.DMA((nbuf,)),
            pltpu.SemaphoreType.DMA((nbuf,)),
        ],
    )
    def k(tbl_hbm, idx_hbm, int_hbm, wl_hbm, out_hbm, idx_v, int_v, rows_v, wl_v, gsem, gsem2, wsem, wsem2):
        wid = lax.axis_index("s") * nc + lax.axis_index("c")
        base = wid * per_w
        pltpu.sync_copy(idx_hbm.at[pl.ds(base, per_w)], idx_v)
        pltpu.sync_copy(int_hbm.at[pl.ds(base, per_w)], int_v)
        pltpu.sync_copy(wl_hbm, wl_v)
        wvals = [wl_v[pl.ds(c * LANES, LANES)] for c in range(D // LANES)]

        hc = ch // 2

        class _Pair:
            def __init__(self, a, b):
                self.a, self.b = a, b

            def start(self):
                self.a.start()
                self.b.start()

            def wait(self):
                self.a.wait()
                self.b.wait()

        def g_copy(t, bs):
            return _Pair(
                pltpu.make_async_copy(
                    tbl_hbm.at[idx_v.at[pl.ds(t * ch, hc)]],
                    rows_v.at[bs, pl.ds(0, hc)],
                    gsem.at[bs],
                ),
                pltpu.make_async_copy(
                    tbl_hbm.at[idx_v.at[pl.ds(t * ch + hc, hc)]],
                    rows_v.at[bs, pl.ds(hc, hc)],
                    gsem2.at[bs],
                ),
            )

        def w_copy(t, bs):
            return _Pair(
                pltpu.make_async_copy(
                    rows_v.at[bs, pl.ds(0, hc)],
                    out_hbm.at[pl.ds(base + t * ch, hc)],
                    wsem.at[bs],
                ),
                pltpu.make_async_copy(
                    rows_v.at[bs, pl.ds(hc, hc)],
                    out_hbm.at[pl.ds(base + t * ch + hc, hc)],
                    wsem2.at[bs],
                ),
            )

        la = 3
        for p in range(la):
            g_copy(p, p).start()

        def outer(g, carry):
            for bs in range(nbuf):
                t = g * nbuf + bs
                bg = (bs + la) % nbuf

                @pl.when(jnp.logical_and(t >= nbuf - la, t + la < nchunk))
                def _():
                    w_copy(t + la - nbuf, bg).wait()

                @pl.when(t + la < nchunk)
                def _():
                    g_copy(t + la, bg).start()

                g_copy(t, bs).wait()
                rows_b = rows_v.at[bs]

                def grp(j, c2, t=t, rows_b=rows_b):
                    iv16 = int_v[pl.ds(t * ch + j * LANES, LANES)]
                    for r in range(LANES):
                        iv = iv16[r]
                        for c in range(D // LANES):
                            plsc.addupdate(
                                rows_b.at[j * LANES + r, pl.ds(c * LANES, LANES)],
                                iv * wvals[c],
                            )
                    return c2

                lax.fori_loop(0, ngrp, grp, 0)
                w_copy(t, bs).start()
            return carry

        lax.fori_loop(0, nouter, outer, 0)
        for bs in range(nbuf):
            w_copy(nchunk - nbuf + bs, bs).wait()

    return k(table_t, idx_flat, int_flat, w_last)


def kernel(mz_batch, int_batch, mz_table, W, b):
    bsz, seq = mz_batch.shape
    w1t = jnp.transpose(W[:, :D])
    w_last = W[:, D]
    table_t = _transform_table(mz_table, w1t, b.reshape(1, D))
    idx_flat = jnp.transpose(mz_batch).reshape(-1).astype(jnp.int32)
    int_flat = jnp.transpose(int_batch).reshape(-1)
    out = _gather_combine(table_t, idx_flat, int_flat, w_last)
    return out.reshape(seq, bsz, D).transpose(1, 0, 2)


# speedup vs baseline: 1.0002x; 1.0002x over previous
import functools

import jax
import jax.numpy as jnp
from jax import lax
from jax.experimental import pallas as pl
from jax.experimental.pallas import tpu as pltpu
from jax.experimental.pallas import tpu_sc as plsc

D = 128
LANES = 16


def _transform_table(mz_table, w1t, b_row):
    v = mz_table.shape[0]
    blk = 16384

    def body(tbl_ref, w_ref, b_ref, out_ref):
        out_ref[...] = (
            jnp.dot(tbl_ref[...], w_ref[...], preferred_element_type=jnp.float32)
            + b_ref[...]
        )

    return pl.pallas_call(
        body,
        grid=(pl.cdiv(v, blk),),
        in_specs=[
            pl.BlockSpec((blk, D), lambda i: (i, 0)),
            pl.BlockSpec((D, D), lambda i: (0, 0)),
            pl.BlockSpec((1, D), lambda i: (0, 0)),
        ],
        out_specs=pl.BlockSpec((blk, D), lambda i: (i, 0)),
        out_shape=jax.ShapeDtypeStruct((v, D), jnp.float32),
    )(mz_table, w1t, b_row)


def _gather_combine(table_t, idx_flat, int_flat, w_last):
    n = idx_flat.shape[0]
    info = plsc.get_sparse_core_info()
    nc, ns = info.num_cores, info.num_subcores
    nw = nc * ns
    per_w = n // nw
    ch = 128
    nbuf = 5
    nchunk = per_w // ch
    nouter = nchunk // nbuf
    ngrp = ch // LANES

    mesh = plsc.VectorSubcoreMesh(core_axis_name="c", subcore_axis_name="s")

    @functools.partial(
        pl.kernel,
        mesh=mesh,
        out_type=jax.ShapeDtypeStruct((n, D), jnp.float32),
        scratch_types=[
            pltpu.VMEM((per_w,), jnp.int32),
            pltpu.VMEM((per_w,), jnp.float32),
            pltpu.VMEM((nbuf, ch, D), jnp.float32),
            pltpu.VMEM((D,), jnp.float32),
            pltpu.SemaphoreType.DMA((nbuf,)),
            pltpu.SemaphoreType.DMA((nbuf,)),
            pltpu.SemaphoreType.DMA((nbuf,)),
        ],
    )
    def k(tbl_hbm, idx_hbm, int_hbm, wl_hbm, out_hbm, idx_v, int_v, rows_v, wl_v, gsem, gsem2, wsem):
        wid = lax.axis_index("s") * nc + lax.axis_index("c")
        base = wid * per_w
        pltpu.sync_copy(idx_hbm.at[pl.ds(base, per_w)], idx_v)
        pltpu.sync_copy(int_hbm.at[pl.ds(base, per_w)], int_v)
        pltpu.sync_copy(wl_hbm, wl_v)
        wvals = [wl_v[pl.ds(c * LANES, LANES)] for c in range(D // LANES)]

        hc = ch // 2

        class _Pair:
            def __init__(self, a, b):
                self.a, self.b = a, b

            def start(self):
                self.a.start()
                self.b.start()

            def wait(self):
                self.a.wait()
                self.b.wait()

        def g_copy(t, bs):
            return _Pair(
                pltpu.make_async_copy(
                    tbl_hbm.at[idx_v.at[pl.ds(t * ch, hc)]],
                    rows_v.at[bs, pl.ds(0, hc)],
                    gsem.at[bs],
                ),
                pltpu.make_async_copy(
                    tbl_hbm.at[idx_v.at[pl.ds(t * ch + hc, hc)]],
                    rows_v.at[bs, pl.ds(hc, hc)],
                    gsem2.at[bs],
                ),
            )

        def w_copy(t, bs):
            return pltpu.make_async_copy(
                rows_v.at[bs], out_hbm.at[pl.ds(base + t * ch, ch)], wsem.at[bs]
            )

        la = 3
        for p in range(la):
            g_copy(p, p).start()

        def outer(g, carry):
            for bs in range(nbuf):
                t = g * nbuf + bs
                bg = (bs + la) % nbuf

                @pl.when(jnp.logical_and(t >= nbuf - la, t + la < nchunk))
                def _():
                    w_copy(t + la - nbuf, bg).wait()

                @pl.when(t + la < nchunk)
                def _():
                    g_copy(t + la, bg).start()

                g_copy(t, bs).wait()
                rows_b = rows_v.at[bs]

                def grp(j, c2, t=t, rows_b=rows_b):
                    iv16 = int_v[pl.ds(t * ch + j * LANES, LANES)]
                    for r in range(LANES):
                        iv = iv16[r]
                        for c in range(D // LANES):
                            plsc.addupdate(
                                rows_b.at[j * LANES + r, pl.ds(c * LANES, LANES)],
                                iv * wvals[c],
                            )
                    return c2

                lax.fori_loop(0, ngrp, grp, 0)
                w_copy(t, bs).start()
            return carry

        lax.fori_loop(0, nouter, outer, 0)
        for bs in range(nbuf):
            w_copy(nchunk - nbuf + bs, bs).wait()

    return k(table_t, idx_flat, int_flat, w_last)


def kernel(mz_batch, int_batch, mz_table, W, b):
    bsz, seq = mz_batch.shape
    w1t = jnp.transpose(W[:, :D])
    w_last = W[:, D]
    table_t = _transform_table(mz_table, w1t, b.reshape(1, D))
    idx_flat = jnp.transpose(mz_batch).reshape(-1).astype(jnp.int32)
    int_flat = jnp.transpose(int_batch).reshape(-1)
    out = _gather_combine(table_t, idx_flat, int_flat, w_last)
    return out.reshape(seq, bsz, D).transpose(1, 0, 2)
